# Initial kernel scaffold; baseline (speedup 1.0000x reference)
import functools

import jax
import jax.numpy as jnp
from jax import lax
from jax.experimental import pallas as pl
from jax.experimental.pallas import tpu as pltpu
from jax.experimental.pallas import tpu_sc as plsc

_K = 8192
_C = 32
_BETA = 0.25
_TT = 1024
_KC = 2048

_NC = 2
_NS = 16
_NW = _NC * _NS


def _argmin_body(z_ref, cw_ref, idx_ref, loss_ref):
    i = pl.program_id(0)
    z = z_ref[...]
    z2 = jnp.sum(z * z, axis=1, keepdims=True)
    bestd = jnp.full((_TT, 1), jnp.inf, dtype=jnp.float32)
    besti = jnp.zeros((_TT, 1), dtype=jnp.int32)
    for c in range(_K // _KC):
        cwc = cw_ref[pl.ds(c * _KC, _KC), :]
        e2 = jnp.sum(cwc * cwc, axis=1)[None, :]
        g = lax.dot_general(z, cwc, (((1,), (1,)), ((), ())),
                            preferred_element_type=jnp.float32)
        d = z2 - 2.0 * g + e2
        dmin = jnp.min(d, axis=1, keepdims=True)
        darg = jnp.argmin(d, axis=1).astype(jnp.int32)[:, None] + c * _KC
        upd = dmin < bestd
        bestd = jnp.where(upd, dmin, bestd)
        besti = jnp.where(upd, darg, besti)
    idx_ref[...] = besti

    @pl.when(i == 0)
    def _init():
        loss_ref[...] = jnp.zeros_like(loss_ref)

    loss_ref[...] += jnp.sum(bestd, axis=(0, 1), keepdims=True)


def _tc_argmin(z, cw):
    n = z.shape[0]
    return pl.pallas_call(
        _argmin_body,
        grid=(n // _TT,),
        in_specs=[
            pl.BlockSpec((_TT, _C), lambda i: (i, 0)),
            pl.BlockSpec((_K, _C), lambda i: (0, 0)),
        ],
        out_specs=[
            pl.BlockSpec((_TT, 1), lambda i: (i, 0)),
            pl.BlockSpec((1, 1), lambda i: (0, 0)),
        ],
        out_shape=[
            jax.ShapeDtypeStruct((n, 1), jnp.int32),
            jax.ShapeDtypeStruct((1, 1), jnp.float32),
        ],
    )(z, cw)


def _sc_gather(table, idx):
    b = idx.shape[0]
    bpw = b // _NW
    mesh = plsc.VectorSubcoreMesh(core_axis_name="c", subcore_axis_name="s")

    @functools.partial(
        pl.kernel, mesh=mesh,
        out_type=jax.ShapeDtypeStruct((b, _C), jnp.float32),
        scratch_types=[
            pltpu.VMEM((bpw,), jnp.int32),
            pltpu.VMEM((bpw, _C), jnp.float32),
            pltpu.SemaphoreType.DMA,
        ],
    )
    def k(table_hbm, idx_hbm, out_hbm, idx_v, rows_v, sem):
        wid = lax.axis_index("s") * _NC + lax.axis_index("c")
        base = wid * bpw
        pltpu.sync_copy(idx_hbm.at[pl.ds(base, bpw)], idx_v)
        pltpu.async_copy(table_hbm.at[idx_v], rows_v, sem).wait()
        pltpu.sync_copy(rows_v, out_hbm.at[pl.ds(base, bpw)])

    return k(table, idx)


def kernel(z_e, codebook):
    b, n, c = z_e.shape
    z = z_e.reshape(-1, c).astype(jnp.float32)
    cw = codebook.astype(jnp.float32)
    idx2, loss_sum = _tc_argmin(z, cw)
    idx = idx2.reshape(-1)
    z_q = _sc_gather(cw, idx)
    vq_loss = loss_sum[0, 0] * (_BETA / (z.shape[0] * c))
    return z_q.reshape(b, n, c), idx.reshape(b, n), vq_loss


# speedup vs baseline: 1.1160x; 1.1160x over previous
import functools

import jax
import jax.numpy as jnp
from jax import lax
from jax.experimental import pallas as pl
from jax.experimental.pallas import tpu as pltpu
from jax.experimental.pallas import tpu_sc as plsc

_K = 8192
_C = 32
_BETA = 0.25
_TT = 1024
_KC = 2048

_NC = 2
_NS = 16
_NW = _NC * _NS


def _argmin_body(z_ref, cw_ref, idx_ref, loss_ref):
    i = pl.program_id(0)
    z = z_ref[...]
    z2 = jnp.sum(z * z, axis=1, keepdims=True)
    bestd = jnp.full((_TT, 1), jnp.inf, dtype=jnp.float32)
    besti = jnp.zeros((_TT, 1), dtype=jnp.int32)
    lane = lax.broadcasted_iota(jnp.int32, (_TT, _KC), 1)
    for c in range(_K // _KC):
        cwc = cw_ref[pl.ds(c * _KC, _KC), :]
        e2 = jnp.sum(cwc * cwc, axis=1)[None, :]
        g = lax.dot_general(z, cwc, (((1,), (1,)), ((), ())),
                            preferred_element_type=jnp.float32)
        d = z2 - 2.0 * g + e2
        dmin = jnp.min(d, axis=1, keepdims=True)
        darg = jnp.min(jnp.where(d == dmin, lane, _KC),
                       axis=1, keepdims=True) + c * _KC
        upd = dmin < bestd
        dmin_b = dmin.astype(jnp.bfloat16).astype(jnp.float32)
        bestd = jnp.where(upd, dmin_b, bestd)
        besti = jnp.where(upd, darg, besti)
    idx_ref[...] = besti

    @pl.when(i == 0)
    def _init():
        loss_ref[...] = jnp.zeros_like(loss_ref)

    loss_ref[...] += jnp.sum(bestd, axis=(0, 1), keepdims=True)


def _tc_argmin(z, cw):
    n = z.shape[0]
    return pl.pallas_call(
        _argmin_body,
        grid=(n // _TT,),
        in_specs=[
            pl.BlockSpec((_TT, _C), lambda i: (i, 0)),
            pl.BlockSpec((_K, _C), lambda i: (0, 0)),
        ],
        out_specs=[
            pl.BlockSpec((_TT, 1), lambda i: (i, 0)),
            pl.BlockSpec((1, 1), lambda i: (0, 0)),
        ],
        out_shape=[
            jax.ShapeDtypeStruct((n, 1), jnp.int32),
            jax.ShapeDtypeStruct((1, 1), jnp.float32),
        ],
    )(z, cw)


def _sc_gather(table, idx):
    b = idx.shape[0]
    bpw = b // _NW
    mesh = plsc.VectorSubcoreMesh(core_axis_name="c", subcore_axis_name="s")

    @functools.partial(
        pl.kernel, mesh=mesh,
        out_type=jax.ShapeDtypeStruct((b, _C), jnp.float32),
        scratch_types=[
            pltpu.VMEM((bpw,), jnp.int32),
            pltpu.VMEM((bpw, _C), jnp.float32),
            pltpu.SemaphoreType.DMA,
        ],
        compiler_params=pltpu.CompilerParams(use_tc_tiling_on_sc=False),
    )
    def k(table_hbm, idx_hbm, out_hbm, idx_v, rows_v, sem):
        wid = lax.axis_index("s") * _NC + lax.axis_index("c")
        base = wid * bpw
        pltpu.sync_copy(idx_hbm.at[pl.ds(base, bpw)], idx_v)
        pltpu.async_copy(table_hbm.at[idx_v], rows_v, sem).wait()
        pltpu.sync_copy(rows_v, out_hbm.at[pl.ds(base, bpw)])

    return k(table, idx)


def kernel(z_e, codebook):
    b, n, c = z_e.shape
    z = z_e.reshape(-1, c).astype(jnp.float32)
    cw = codebook.astype(jnp.float32)
    idx2, loss_sum = _tc_argmin(z, cw)
    idx = idx2.reshape(-1)
    z_q = _sc_gather(cw, idx)
    vq_loss = loss_sum[0, 0] * (_BETA / (z.shape[0] * c))
    return z_q.reshape(b, n, c), idx.reshape(b, n), vq_loss


# speedup vs baseline: 1.2753x; 1.1428x over previous
import functools

import jax
import jax.numpy as jnp
from jax import lax
from jax.experimental import pallas as pl
from jax.experimental.pallas import tpu as pltpu
from jax.experimental.pallas import tpu_sc as plsc

_K = 8192
_C = 32
_BETA = 0.25
_TT = 2048
_KC = 2048

_NC = 2
_NS = 16
_NW = _NC * _NS


def _argmin_body(z_ref, cw_ref, idx_ref, loss_ref):
    i = pl.program_id(0)
    z = z_ref[...]
    z2 = jnp.sum(z * z, axis=1, keepdims=True)
    bestd = jnp.full((_TT, 1), jnp.inf, dtype=jnp.float32)
    besti = jnp.zeros((_TT, 1), dtype=jnp.float32)
    lane = lax.broadcasted_iota(jnp.int32, (_TT, _KC), 1).astype(jnp.float32)
    for c in range(_K // _KC):
        cwc = cw_ref[pl.ds(c * _KC, _KC), :]
        e2 = jnp.sum(cwc * cwc, axis=1)[None, :]
        g = lax.dot_general(z, cwc, (((1,), (1,)), ((), ())),
                            preferred_element_type=jnp.float32)
        d = z2 - 2.0 * g + e2
        dmin = jnp.min(d, axis=1, keepdims=True)
        darg = jnp.min(jnp.where(d == dmin, lane, float(_KC)),
                       axis=1, keepdims=True) + float(c * _KC)
        upd = dmin < bestd
        dmin_b = dmin.astype(jnp.bfloat16).astype(jnp.float32)
        bestd = jnp.where(upd, dmin_b, bestd)
        besti = jnp.where(upd, darg, besti)
    idx_ref[...] = besti.astype(jnp.int32)

    @pl.when(i == 0)
    def _init():
        loss_ref[...] = jnp.zeros_like(loss_ref)

    loss_ref[...] += jnp.sum(bestd, axis=(0, 1), keepdims=True)


def _tc_argmin(z, cw):
    n = z.shape[0]
    return pl.pallas_call(
        _argmin_body,
        grid=(n // _TT,),
        in_specs=[
            pl.BlockSpec((_TT, _C), lambda i: (i, 0)),
            pl.BlockSpec((_K, _C), lambda i: (0, 0)),
        ],
        out_specs=[
            pl.BlockSpec((_TT, 1), lambda i: (i, 0)),
            pl.BlockSpec((1, 1), lambda i: (0, 0)),
        ],
        out_shape=[
            jax.ShapeDtypeStruct((n, 1), jnp.int32),
            jax.ShapeDtypeStruct((1, 1), jnp.float32),
        ],
    )(z, cw)


def _sc_gather(table, idx):
    b = idx.shape[0]
    bpw = b // _NW
    mesh = plsc.VectorSubcoreMesh(core_axis_name="c", subcore_axis_name="s")

    @functools.partial(
        pl.kernel, mesh=mesh,
        out_type=jax.ShapeDtypeStruct((b, _C), jnp.float32),
        scratch_types=[
            pltpu.VMEM((bpw,), jnp.int32),
            pltpu.VMEM((bpw, _C), jnp.float32),
            pltpu.SemaphoreType.DMA,
        ],
        compiler_params=pltpu.CompilerParams(use_tc_tiling_on_sc=False),
    )
    def k(table_hbm, idx_hbm, out_hbm, idx_v, rows_v, sem):
        wid = lax.axis_index("s") * _NC + lax.axis_index("c")
        base = wid * bpw
        pltpu.sync_copy(idx_hbm.at[pl.ds(base, bpw)], idx_v)
        pltpu.async_copy(table_hbm.at[idx_v], rows_v, sem).wait()
        pltpu.sync_copy(rows_v, out_hbm.at[pl.ds(base, bpw)])

    return k(table, idx)


def kernel(z_e, codebook):
    b, n, c = z_e.shape
    z = z_e.reshape(-1, c).astype(jnp.float32)
    cw = codebook.astype(jnp.float32)
    idx2, loss_sum = _tc_argmin(z, cw)
    idx = idx2.reshape(-1)
    z_q = _sc_gather(cw, idx)
    vq_loss = loss_sum[0, 0] * (_BETA / (z.shape[0] * c))
    return z_q.reshape(b, n, c), idx.reshape(b, n), vq_loss
